# baseline (speedup 1.0000x reference)
import functools

import jax
import jax.numpy as jnp
from jax import lax
from jax.experimental import pallas as pl
from jax.experimental.pallas import tpu as pltpu
from jax.experimental.pallas import tpu_sc as plsc


def _make_sc_gather(V, D, B):
    info = plsc.get_sparse_core_info()
    NC, NS = info.num_cores, info.num_subcores
    NW = NC * NS
    assert B % NW == 0
    b_per_w = B // NW
    CH = 32
    assert b_per_w % CH == 0
    n_chunks = b_per_w // CH

    mesh = plsc.VectorSubcoreMesh(core_axis_name="c", subcore_axis_name="s")

    @functools.partial(
        pl.kernel,
        mesh=mesh,
        out_type=jax.ShapeDtypeStruct((B, D), jnp.float32),
        scratch_types=[
            pltpu.VMEM((b_per_w,), jnp.int32),
            pltpu.VMEM((CH, D), jnp.float32),
            pltpu.VMEM((CH, D), jnp.float32),
            pltpu.SemaphoreType.DMA,
            pltpu.SemaphoreType.DMA,
        ],
    )
    def gather_kernel(table_hbm, ids_hbm, out_hbm, idx_v, buf0, buf1, g0, g1):
        wid = lax.axis_index("s") * NC + lax.axis_index("c")
        base = wid * b_per_w
        pltpu.sync_copy(ids_hbm.at[pl.ds(base, b_per_w)], idx_v)
        bufs = (buf0, buf1)
        sems = (g0, g1)
        pltpu.async_copy(table_hbm.at[idx_v.at[pl.ds(0, CH)]], bufs[0], sems[0])
        for c in range(n_chunks):
            cur = bufs[c % 2]
            if c + 1 < n_chunks:
                pltpu.async_copy(
                    table_hbm.at[idx_v.at[pl.ds((c + 1) * CH, CH)]],
                    bufs[(c + 1) % 2],
                    sems[(c + 1) % 2],
                )
            pltpu.make_async_copy(
                table_hbm.at[idx_v.at[pl.ds(c * CH, CH)]], cur, sems[c % 2]
            ).wait()
            pltpu.sync_copy(cur, out_hbm.at[pl.ds(base + c * CH, CH)])

    return gather_kernel


def _mm_body(x_ref, w_ref, b_ref, o_ref):
    o_ref[...] = (
        lax.dot_general(
            x_ref[...].astype(jnp.bfloat16),
            w_ref[...].astype(jnp.bfloat16),
            dimension_numbers=(((1,), (1,)), ((), ())),
            preferred_element_type=jnp.float32,
        )
        + b_ref[...]
    )


def _make_tc_matmul(B, D, BB=1024):
    return pl.pallas_call(
        _mm_body,
        grid=(B // BB,),
        in_specs=[
            pl.BlockSpec((BB, D), lambda i: (i, 0)),
            pl.BlockSpec((D, D), lambda i: (0, 0)),
            pl.BlockSpec((1, D), lambda i: (0, 0)),
        ],
        out_specs=pl.BlockSpec((BB, D), lambda i: (i, 0)),
        out_shape=jax.ShapeDtypeStruct((B, D), jnp.float32),
    )


def kernel(user_ids, embedding_table, W, b):
    B = user_ids.shape[0]
    V, D = embedding_table.shape
    ids = user_ids.astype(jnp.int32)
    gathered = _make_sc_gather(V, D, B)(embedding_table, ids)
    return _make_tc_matmul(B, D)(gathered, W, b.reshape(1, D))


# speedup vs baseline: 1.0403x; 1.0403x over previous
import functools

import jax
import jax.numpy as jnp
from jax import lax
from jax.experimental import pallas as pl
from jax.experimental.pallas import tpu as pltpu
from jax.experimental.pallas import tpu_sc as plsc


def _make_sc_gather(V, D, B):
    info = plsc.get_sparse_core_info()
    NC, NS = info.num_cores, info.num_subcores
    NW = NC * NS
    assert B % NW == 0
    b_per_w = B // NW
    CH = 32
    assert b_per_w % CH == 0
    n_chunks = b_per_w // CH

    mesh = plsc.VectorSubcoreMesh(core_axis_name="c", subcore_axis_name="s")

    @functools.partial(
        pl.kernel,
        mesh=mesh,
        out_type=jax.ShapeDtypeStruct((B, D), jnp.float32),
        scratch_types=[
            pltpu.VMEM((b_per_w,), jnp.int32),
            pltpu.VMEM((CH, D), jnp.float32),
            pltpu.VMEM((CH, D), jnp.float32),
            pltpu.SemaphoreType.DMA,
            pltpu.SemaphoreType.DMA,
        ],
    )
    def gather_kernel(table_hbm, ids_hbm, out_hbm, idx_v, buf0, buf1, g0, g1):
        wid = lax.axis_index("s") * NC + lax.axis_index("c")
        base = wid * b_per_w
        pltpu.sync_copy(ids_hbm.at[pl.ds(base, b_per_w)], idx_v)
        bufs = (buf0, buf1)
        sems = (g0, g1)
        pltpu.async_copy(table_hbm.at[idx_v.at[pl.ds(0, CH)]], bufs[0], sems[0])
        for c in range(n_chunks):
            cur = bufs[c % 2]
            if c + 1 < n_chunks:
                pltpu.async_copy(
                    table_hbm.at[idx_v.at[pl.ds((c + 1) * CH, CH)]],
                    bufs[(c + 1) % 2],
                    sems[(c + 1) % 2],
                )
            pltpu.make_async_copy(
                table_hbm.at[idx_v.at[pl.ds(c * CH, CH)]], cur, sems[c % 2]
            ).wait()
            pltpu.sync_copy(cur, out_hbm.at[pl.ds(base + c * CH, CH)])

    return gather_kernel


def _mm_compute(x_ref, w_ref, b_ref, o_ref):
    o_ref[...] = (
        lax.dot_general(
            x_ref[...],
            w_ref[...],
            dimension_numbers=(((1,), (1,)), ((), ())),
            preferred_element_type=jnp.float32,
        )
        + b_ref[...]
    )


def _mm_body(x_ref, w_ref, b_ref, o_ref):
    _mm_compute(x_ref, w_ref, b_ref, o_ref)


def _mm_body_aliased(y_ref, x_ref, w_ref, b_ref, o_ref):
    del y_ref
    _mm_compute(x_ref, w_ref, b_ref, o_ref)


def _make_tc_matmul_chunk(B, D, CB, off_rows, aliased, BB=1024):
    base_blk = off_rows // BB
    xwb_specs = [
        pl.BlockSpec((BB, D), lambda j: (j, 0)),
        pl.BlockSpec((D, D), lambda j: (0, 0)),
        pl.BlockSpec((1, D), lambda j: (0, 0)),
    ]
    if aliased:
        in_specs = [pl.BlockSpec(memory_space=pl.ANY)] + xwb_specs
        body = _mm_body_aliased
    else:
        in_specs = xwb_specs
        body = _mm_body
    return pl.pallas_call(
        body,
        grid=(CB // BB,),
        in_specs=in_specs,
        out_specs=pl.BlockSpec((BB, D), lambda j: (base_blk + j, 0)),
        out_shape=jax.ShapeDtypeStruct((B, D), jnp.float32),
        input_output_aliases={0: 0} if aliased else {},
    )


def kernel(user_ids, embedding_table, W, b):
    B = user_ids.shape[0]
    V, D = embedding_table.shape
    ids = user_ids.astype(jnp.int32)
    b2 = b.reshape(1, D)
    NCH = 4
    CB = B // NCH
    gather = _make_sc_gather(V, D, CB)
    chunks = [
        gather(embedding_table, lax.slice(ids, (i * CB,), ((i + 1) * CB,)))
        for i in range(NCH)
    ]
    y = _make_tc_matmul_chunk(B, D, CB, 0, aliased=False)(chunks[0], W, b2)
    for i in range(1, NCH):
        y = _make_tc_matmul_chunk(B, D, CB, i * CB, aliased=True)(
            y, chunks[i], W, b2
        )
    return y
